# baseline (device time: 62531 ns/iter reference)
import jax
import jax.numpy as jnp
from jax import lax
from jax.experimental import pallas as pl
from jax.experimental.pallas import tpu as pltpu

NZ = 4
T = 256
QB = T // 4
FB = QB // 2
D = 4096
N_FULL = NZ * D
NH = NZ - 1
S = 4
SD = D // S


def kernel(x, W):
    def body(x_ref, w_ref, out_ref, g_ref,
             zs, zr, xds, xdr, yds, ydr, xfs, xfr, yfs, yfr):
        my_x = lax.axis_index("x")
        my_y = lax.axis_index("y")
        my_z = lax.axis_index("z")
        left = (my_z - 1) % NZ
        right = (my_z + 1) % NZ
        r = 2 * my_x + my_y
        r_x = 2 * (1 - my_x) + my_y
        r_y = 2 * my_x + (1 - my_y)

        barrier_sem = pltpu.get_barrier_semaphore()
        for dev in ((my_x, my_y, left), (my_x, my_y, right),
                    (1 - my_x, my_y, my_z), (my_x, 1 - my_y, my_z)):
            pl.semaphore_signal(
                barrier_sem, inc=1,
                device_id=dev, device_id_type=pl.DeviceIdType.MESH,
            )
        pl.semaphore_wait(barrier_sem, 4)

        def desc(rows, nrows, cols, send_sem, recv_sem, dev):
            return pltpu.make_async_remote_copy(
                src_ref=g_ref.at[pl.ds(rows, nrows), pl.ds(cols, SD)],
                dst_ref=g_ref.at[pl.ds(rows, nrows), pl.ds(cols, SD)],
                send_sem=send_sem,
                recv_sem=recv_sem,
                device_id=dev,
                device_id_type=pl.DeviceIdType.MESH,
            )

        x_nbr = (1 - my_x, my_y, my_z)
        y_nbr = (my_x, 1 - my_y, my_z)

        def ring_desc(h, s):
            c = (my_z - h) % NZ
            return desc(r * QB, QB, c * D + s * SD, zs.at[h, s],
                        zr.at[h, s], (my_x, my_y, right))

        def xdir_desc(h, s):
            c = (my_z - h - 1) % NZ
            return desc(r * QB, QB, c * D + s * SD, xds.at[h, s],
                        xdr.at[h, s], x_nbr)

        def ydir_desc(h, s):
            c = (my_z - h - 1) % NZ
            return desc(r * QB, QB, c * D + s * SD, yds.at[h, s],
                        ydr.at[h, s], y_nbr)

        def xfwd_desc(h, s):
            c = (my_z - h - 1) % NZ
            return desc(r_y * QB, FB, c * D + s * SD, xfs.at[h, s],
                        xfr.at[h, s], x_nbr)

        def yfwd_desc(h, s):
            c = (my_z - h - 1) % NZ
            return desc(r_x * QB + FB, FB, c * D + s * SD, yfs.at[h, s],
                        yfr.at[h, s], y_nbr)

        x_bf = x_ref[...].astype(jnp.bfloat16)
        ring = {}
        for s in range(S):
            logits = jnp.dot(x_bf,
                             w_ref[:, s * SD:(s + 1) * SD].astype(
                                 jnp.bfloat16),
                             preferred_element_type=jnp.float32)
            g_ref[:, pl.ds(my_z * D + s * SD, SD)] = (
                jnp.exp(logits).astype(jnp.bfloat16))
            ring[(0, s)] = ring_desc(0, s)
            ring[(0, s)].start()

        xdir, ydir, xfwd, yfwd = {}, {}, {}, {}
        for h in range(NH):
            for s in range(S):
                ring[(h, s)].wait_recv()
                if h + 1 < NH:
                    ring[(h + 1, s)] = ring_desc(h + 1, s)
                    ring[(h + 1, s)].start()
                xdir[(h, s)] = xdir_desc(h, s)
                xdir[(h, s)].start()
                ydir[(h, s)] = ydir_desc(h, s)
                ydir[(h, s)].start()
                if (h, s) != (0, 0):
                    ph, ps = (h, s - 1) if s > 0 else (h - 1, S - 1)
                    xdir[(ph, ps)].wait_recv()
                    yfwd[(ph, ps)] = yfwd_desc(ph, ps)
                    yfwd[(ph, ps)].start()
                    ydir[(ph, ps)].wait_recv()
                    xfwd[(ph, ps)] = xfwd_desc(ph, ps)
                    xfwd[(ph, ps)].start()

        xdir[(NH - 1, S - 1)].wait_recv()
        yfwd[(NH - 1, S - 1)] = yfwd_desc(NH - 1, S - 1)
        yfwd[(NH - 1, S - 1)].start()
        ydir[(NH - 1, S - 1)].wait_recv()
        xfwd[(NH - 1, S - 1)] = xfwd_desc(NH - 1, S - 1)
        xfwd[(NH - 1, S - 1)].start()

        acc = jnp.sum(
            g_ref[:, pl.ds(my_z * D, D)].astype(jnp.float32),
            axis=1, keepdims=True)
        for h in range(NH):
            for s in range(S):
                xfwd[(h, s)].wait_recv()
                yfwd[(h, s)].wait_recv()
            c = (my_z - h - 1) % NZ
            acc = acc + jnp.sum(
                g_ref[:, pl.ds(c * D, D)].astype(jnp.float32),
                axis=1, keepdims=True)

        for ds in (ring, xdir, ydir, xfwd, yfwd):
            for d in ds.values():
                d.wait_send()
        inv = 1.0 / acc
        for c in range(NZ):
            out_ref[:, c * D:(c + 1) * D] = (
                g_ref[:, c * D:(c + 1) * D].astype(jnp.float32) * inv)

    return pl.pallas_call(
        body,
        out_shape=jax.ShapeDtypeStruct((T, N_FULL), jnp.float32),
        in_specs=[
            pl.BlockSpec(memory_space=pltpu.VMEM),
            pl.BlockSpec(memory_space=pltpu.VMEM),
        ],
        out_specs=pl.BlockSpec(memory_space=pltpu.VMEM),
        scratch_shapes=[pltpu.VMEM((T, N_FULL), jnp.bfloat16)]
        + [pltpu.SemaphoreType.DMA((NH, S))] * 10,
        compiler_params=pltpu.CompilerParams(collective_id=0),
    )(x, W)
